# baseline (device time: 28352 ns/iter reference)
import jax
import jax.numpy as jnp
from jax import lax
from jax.experimental import pallas as pl
from jax.experimental.pallas import tpu as pltpu

N_DEV = 16
B, SQ, SKV = 2, 256, 256
H_LOCAL, DH = 4, 64
D_MODEL = 512
ROWS = B * SQ
COLS = D_MODEL
CH = ROWS // N_DEV


def _fused_body(
    x_ref, wq_ref, k_ref, v_ref, wo_ref, out_ref,
    part_ref, stage_ref, comm_ref, rs_send, rs_recv, ag_send, ag_recv,
):
    me = lax.axis_index("i")

    barrier = pltpu.get_barrier_semaphore()
    for off in range(1, N_DEV):
        pl.semaphore_signal(
            barrier,
            inc=1,
            device_id=((me + off) % N_DEV,),
            device_id_type=pl.DeviceIdType.MESH,
        )

    bf = jnp.bfloat16
    qrow = lax.broadcasted_iota(jnp.int32, (SQ, SKV), 0)
    kcol = lax.broadcasted_iota(jnp.int32, (SQ, SKV), 1)
    mask = (jnp.abs(qrow - kcol) <= 128) | (kcol < 32) | (qrow < 32)

    q_all = jax.lax.dot_general(
        x_ref[...], wq_ref[...],
        (((1,), (0,)), ((), ())),
        preferred_element_type=jnp.float32,
    )

    for b in range(B):
        part_b = None
        for h in range(H_LOCAL):
            q_bh = q_all[b * SQ:(b + 1) * SQ, h * DH:(h + 1) * DH].astype(bf)
            k_bh = k_ref[b, :, h, :]
            v_bh = v_ref[b, :, h, :]
            scores = jax.lax.dot_general(
                q_bh, k_bh,
                (((1,), (1,)), ((), ())),
                preferred_element_type=jnp.float32,
            ) * 0.125
            scores = jnp.where(mask, scores, -1e9)
            m = jnp.max(scores, axis=1, keepdims=True)
            e = jnp.exp(scores - m)
            w = (e / jnp.sum(e, axis=1, keepdims=True)).astype(bf)
            ctx = jax.lax.dot_general(
                w, v_bh,
                (((1,), (0,)), ((), ())),
                preferred_element_type=jnp.float32,
            ).astype(bf)
            contrib = jax.lax.dot_general(
                ctx, wo_ref[h * DH:(h + 1) * DH, :],
                (((1,), (0,)), ((), ())),
                preferred_element_type=jnp.float32,
            )
            part_b = contrib if part_b is None else part_b + contrib
        sl = pl.ds(b * SQ, SQ)
        part_ref[sl, :] = part_b
        stage_ref[sl, :] = part_b.astype(bf)

    pl.semaphore_wait(barrier, N_DEV - 1)
    rs_sends = []
    for off in range(1, N_DEV):
        tgt = (me + off) % N_DEV
        rdma = pltpu.make_async_remote_copy(
            src_ref=stage_ref.at[pl.ds(tgt * CH, CH), :],
            dst_ref=comm_ref.at[me],
            send_sem=rs_send.at[off],
            recv_sem=rs_recv.at[off],
            device_id=(tgt,),
            device_id_type=pl.DeviceIdType.MESH,
        )
        rdma.start()
        rs_sends.append(rdma)

    acc = part_ref[pl.ds(me * CH, CH), :]
    for off in range(1, N_DEV):
        src = (me - off) % N_DEV
        recv = pltpu.make_async_remote_copy(
            src_ref=comm_ref.at[src],
            dst_ref=comm_ref.at[src],
            send_sem=rs_send.at[off],
            recv_sem=rs_recv.at[off],
            device_id=(src,),
            device_id_type=pl.DeviceIdType.MESH,
        )
        recv.wait_recv()
        acc = acc + comm_ref[src].astype(jnp.float32)

    out_ref[pl.ds(me * CH, CH), :] = acc.astype(bf)

    ag_sends = []
    for off in range(1, N_DEV):
        tgt = (me + off) % N_DEV
        rdma = pltpu.make_async_remote_copy(
            src_ref=out_ref.at[pl.ds(me * CH, CH), :],
            dst_ref=out_ref.at[pl.ds(me * CH, CH), :],
            send_sem=ag_send.at[off],
            recv_sem=ag_recv.at[off],
            device_id=(tgt,),
            device_id_type=pl.DeviceIdType.MESH,
        )
        rdma.start()
        ag_sends.append(rdma)

    for off in range(1, N_DEV):
        src = (me - off) % N_DEV
        recv = pltpu.make_async_remote_copy(
            src_ref=out_ref.at[pl.ds(src * CH, CH), :],
            dst_ref=out_ref.at[pl.ds(src * CH, CH), :],
            send_sem=ag_send.at[off],
            recv_sem=ag_recv.at[off],
            device_id=(src,),
            device_id_type=pl.DeviceIdType.MESH,
        )
        recv.wait_recv()

    for rdma in rs_sends + ag_sends:
        rdma.wait_send()


def kernel(x, Wq, K_ext, V_ext, Wo):
    i = lax.axis_index("i")
    bf = jnp.bfloat16

    x2d = x.reshape(ROWS, D_MODEL).astype(bf)
    Kh = lax.dynamic_slice_in_dim(K_ext, i * H_LOCAL, H_LOCAL, axis=2).astype(bf)
    Vh = lax.dynamic_slice_in_dim(V_ext, i * H_LOCAL, H_LOCAL, axis=2).astype(bf)

    out = pl.pallas_call(
        _fused_body,
        out_shape=jax.ShapeDtypeStruct((ROWS, COLS), bf),
        in_specs=[pl.BlockSpec(memory_space=pltpu.VMEM)] * 5,
        out_specs=pl.BlockSpec(memory_space=pltpu.VMEM),
        scratch_shapes=[
            pltpu.VMEM((ROWS, COLS), jnp.float32),
            pltpu.VMEM((ROWS, COLS), bf),
            pltpu.VMEM((N_DEV, CH, COLS), bf),
            pltpu.SemaphoreType.DMA((N_DEV,)),
            pltpu.SemaphoreType.DMA((N_DEV,)),
            pltpu.SemaphoreType.DMA((N_DEV,)),
            pltpu.SemaphoreType.DMA((N_DEV,)),
        ],
        compiler_params=pltpu.CompilerParams(collective_id=0),
    )(x2d, Wq.astype(bf), Kh, Vh, Wo.astype(bf))
    return out.reshape(B, SQ, D_MODEL)


# device time: 25380 ns/iter; 1.1171x vs baseline; 1.1171x over previous
import jax
import jax.numpy as jnp
from jax import lax
from jax.experimental import pallas as pl
from jax.experimental.pallas import tpu as pltpu

N_DEV = 16
B, SQ, SKV = 2, 256, 256
H_LOCAL, DH = 4, 64
D_MODEL = 512
ROWS = B * SQ
COLS = D_MODEL
CH = ROWS // N_DEV


def _allreduce_body(
    p_ref, out_ref, stage_ref, comm_ref, rs_send, rs_recv, ag_send, ag_recv
):
    me = lax.axis_index("i")

    barrier = pltpu.get_barrier_semaphore()
    for off in range(1, N_DEV):
        pl.semaphore_signal(
            barrier,
            inc=1,
            device_id=((me + off) % N_DEV,),
            device_id_type=pl.DeviceIdType.MESH,
        )
    stage_ref[...] = p_ref[...].astype(jnp.bfloat16)
    pl.semaphore_wait(barrier, N_DEV - 1)

    rs_sends = []
    for off in range(1, N_DEV):
        tgt = (me + off) % N_DEV
        rdma = pltpu.make_async_remote_copy(
            src_ref=stage_ref.at[pl.ds(tgt * CH, CH), :],
            dst_ref=comm_ref.at[me],
            send_sem=rs_send.at[off],
            recv_sem=rs_recv.at[off],
            device_id=(tgt,),
            device_id_type=pl.DeviceIdType.MESH,
        )
        rdma.start()
        rs_sends.append(rdma)

    acc = p_ref[pl.ds(me * CH, CH), :]
    for off in range(1, N_DEV):
        src = (me - off) % N_DEV
        recv = pltpu.make_async_remote_copy(
            src_ref=comm_ref.at[src],
            dst_ref=comm_ref.at[src],
            send_sem=rs_send.at[off],
            recv_sem=rs_recv.at[off],
            device_id=(src,),
            device_id_type=pl.DeviceIdType.MESH,
        )
        recv.wait_recv()
        acc = acc + comm_ref[src].astype(jnp.float32)

    out_ref[pl.ds(me * CH, CH), :] = acc.astype(jnp.bfloat16)

    ag_sends = []
    for off in range(1, N_DEV):
        tgt = (me + off) % N_DEV
        rdma = pltpu.make_async_remote_copy(
            src_ref=out_ref.at[pl.ds(me * CH, CH), :],
            dst_ref=out_ref.at[pl.ds(me * CH, CH), :],
            send_sem=ag_send.at[off],
            recv_sem=ag_recv.at[off],
            device_id=(tgt,),
            device_id_type=pl.DeviceIdType.MESH,
        )
        rdma.start()
        ag_sends.append(rdma)

    for off in range(1, N_DEV):
        src = (me - off) % N_DEV
        recv = pltpu.make_async_remote_copy(
            src_ref=out_ref.at[pl.ds(src * CH, CH), :],
            dst_ref=out_ref.at[pl.ds(src * CH, CH), :],
            send_sem=ag_send.at[off],
            recv_sem=ag_recv.at[off],
            device_id=(src,),
            device_id_type=pl.DeviceIdType.MESH,
        )
        recv.wait_recv()

    for rdma in rs_sends + ag_sends:
        rdma.wait_send()


def _alltoall_allreduce(partial):
    return pl.pallas_call(
        _allreduce_body,
        out_shape=jax.ShapeDtypeStruct((ROWS, COLS), jnp.bfloat16),
        in_specs=[pl.BlockSpec(memory_space=pltpu.VMEM)],
        out_specs=pl.BlockSpec(memory_space=pltpu.VMEM),
        scratch_shapes=[
            pltpu.VMEM((ROWS, COLS), jnp.bfloat16),
            pltpu.VMEM((N_DEV, CH, COLS), jnp.bfloat16),
            pltpu.SemaphoreType.DMA((N_DEV,)),
            pltpu.SemaphoreType.DMA((N_DEV,)),
            pltpu.SemaphoreType.DMA((N_DEV,)),
            pltpu.SemaphoreType.DMA((N_DEV,)),
        ],
        compiler_params=pltpu.CompilerParams(collective_id=0),
    )(partial)


def kernel(x, Wq, K_ext, V_ext, Wo):
    i = lax.axis_index("i")
    bf = jnp.bfloat16

    Q = jnp.einsum(
        "bsd,dh->bsh", x.astype(bf), Wq.astype(bf), preferred_element_type=jnp.float32
    ).reshape(B, SQ, H_LOCAL, DH)
    Kh = lax.dynamic_slice_in_dim(K_ext, i * H_LOCAL, H_LOCAL, axis=2)
    Vh = lax.dynamic_slice_in_dim(V_ext, i * H_LOCAL, H_LOCAL, axis=2)

    scores = (
        jnp.einsum(
            "bihd,bjhd->bhij",
            Q.astype(bf),
            Kh.astype(bf),
            preferred_element_type=jnp.float32,
        )
        * 0.125
    )
    qi = jnp.arange(SQ)[:, None]
    ki = jnp.arange(SKV)[None, :]
    mask = (jnp.abs(qi - ki) <= 128) | (ki < 32) | (qi < 32)
    scores = jnp.where(mask[None, None], scores, -1e9)
    m = scores.max(axis=-1, keepdims=True)
    w = jnp.exp(scores - m)
    w = w / w.sum(axis=-1, keepdims=True)

    ctx = jnp.einsum(
        "bhij,bjhd->bihd",
        w.astype(bf),
        Vh.astype(bf),
        preferred_element_type=jnp.float32,
    ).reshape(B, SQ, H_LOCAL * DH)

    partial = jnp.einsum(
        "bsf,fd->bsd",
        ctx.astype(bf),
        Wo.astype(bf),
        preferred_element_type=jnp.float32,
    )

    out = _alltoall_allreduce(partial.reshape(ROWS, COLS))
    return out.reshape(B, SQ, D_MODEL)


# device time: 25334 ns/iter; 1.1191x vs baseline; 1.0018x over previous
import jax
import jax.numpy as jnp
from jax import lax
from jax.experimental import pallas as pl
from jax.experimental.pallas import tpu as pltpu

N_DEV = 16
B, SQ, SKV = 2, 256, 256
H_LOCAL, DH = 4, 64
D_MODEL = 512
ROWS = B * SQ
COLS = D_MODEL
CH = ROWS // N_DEV


def _allreduce_body(
    p_ref, out_ref, stage_ref, comm_ref, rs_send, rs_recv, ag_send, ag_recv
):
    me = lax.axis_index("i")

    barrier = pltpu.get_barrier_semaphore()
    for off in range(1, N_DEV):
        pl.semaphore_signal(
            barrier,
            inc=1,
            device_id=((me + off) % N_DEV,),
            device_id_type=pl.DeviceIdType.MESH,
        )
    stage_ref[...] = p_ref[...].astype(jnp.bfloat16)
    pl.semaphore_wait(barrier, N_DEV - 1)

    rs_sends = []
    for off in range(1, N_DEV):
        tgt = (me + off) % N_DEV
        rdma = pltpu.make_async_remote_copy(
            src_ref=stage_ref.at[pl.ds(tgt * CH, CH), :],
            dst_ref=comm_ref.at[me],
            send_sem=rs_send.at[off],
            recv_sem=rs_recv.at[off],
            device_id=(tgt,),
            device_id_type=pl.DeviceIdType.MESH,
        )
        rdma.start()
        rs_sends.append(rdma)

    acc = p_ref[pl.ds(me * CH, CH), :]
    for off in range(1, N_DEV):
        src = (me - off) % N_DEV
        recv = pltpu.make_async_remote_copy(
            src_ref=comm_ref.at[src],
            dst_ref=comm_ref.at[src],
            send_sem=rs_send.at[off],
            recv_sem=rs_recv.at[off],
            device_id=(src,),
            device_id_type=pl.DeviceIdType.MESH,
        )
        recv.wait_recv()
        acc = acc + comm_ref[src].astype(jnp.float32)

    out_ref[pl.ds(me * CH, CH), :] = acc.astype(jnp.bfloat16)

    ag_sends = []
    for off in range(1, N_DEV):
        tgt = (me + off) % N_DEV
        rdma = pltpu.make_async_remote_copy(
            src_ref=out_ref.at[pl.ds(me * CH, CH), :],
            dst_ref=out_ref.at[pl.ds(me * CH, CH), :],
            send_sem=ag_send.at[off],
            recv_sem=ag_recv.at[off],
            device_id=(tgt,),
            device_id_type=pl.DeviceIdType.MESH,
        )
        rdma.start()
        ag_sends.append(rdma)

    for off in range(1, N_DEV):
        src = (me - off) % N_DEV
        recv = pltpu.make_async_remote_copy(
            src_ref=out_ref.at[pl.ds(src * CH, CH), :],
            dst_ref=out_ref.at[pl.ds(src * CH, CH), :],
            send_sem=ag_send.at[off],
            recv_sem=ag_recv.at[off],
            device_id=(src,),
            device_id_type=pl.DeviceIdType.MESH,
        )
        recv.wait_recv()

    for rdma in rs_sends + ag_sends:
        rdma.wait_send()


def _alltoall_allreduce(partial):
    return pl.pallas_call(
        _allreduce_body,
        out_shape=jax.ShapeDtypeStruct((ROWS, COLS), jnp.bfloat16),
        in_specs=[pl.BlockSpec(memory_space=pltpu.VMEM)],
        out_specs=pl.BlockSpec(memory_space=pltpu.VMEM),
        scratch_shapes=[
            pltpu.VMEM((ROWS, COLS), jnp.bfloat16),
            pltpu.VMEM((N_DEV, CH, COLS), jnp.bfloat16),
            pltpu.SemaphoreType.DMA((N_DEV,)),
            pltpu.SemaphoreType.DMA((N_DEV,)),
            pltpu.SemaphoreType.DMA((N_DEV,)),
            pltpu.SemaphoreType.DMA((N_DEV,)),
        ],
        compiler_params=pltpu.CompilerParams(collective_id=0),
    )(partial)


def kernel(x, Wq, K_ext, V_ext, Wo):
    i = lax.axis_index("i")
    bf = jnp.bfloat16

    Q = jnp.einsum(
        "bsd,dh->bsh", x.astype(bf), Wq.astype(bf), preferred_element_type=jnp.float32
    ).reshape(B, SQ, H_LOCAL, DH)
    Kh = lax.dynamic_slice_in_dim(K_ext, i * H_LOCAL, H_LOCAL, axis=2)
    Vh = lax.dynamic_slice_in_dim(V_ext, i * H_LOCAL, H_LOCAL, axis=2)

    scores = jnp.einsum(
        "bihd,bjhd->bhij",
        (Q * 0.125).astype(bf),
        Kh.astype(bf),
        preferred_element_type=jnp.float32,
    )
    qi = jnp.arange(SQ)[:, None]
    ki = jnp.arange(SKV)[None, :]
    mask = (jnp.abs(qi - ki) <= 128) | (ki < 32) | (qi < 32)
    w = jnp.exp(jnp.where(mask[None, None], scores, -1e9))
    w = w / w.sum(axis=-1, keepdims=True)

    ctx = jnp.einsum(
        "bhij,bjhd->bihd",
        w.astype(bf),
        Vh.astype(bf),
        preferred_element_type=jnp.float32,
    ).reshape(B, SQ, H_LOCAL * DH)

    partial = jnp.einsum(
        "bsf,fd->bsd",
        ctx.astype(bf),
        Wo.astype(bf),
        preferred_element_type=jnp.float32,
    )

    out = _alltoall_allreduce(partial.reshape(ROWS, COLS))
    return out.reshape(B, SQ, D_MODEL)
